# baseline (speedup 1.0000x reference)
import functools

import jax
import jax.numpy as jnp
import numpy as np
from jax import lax
from jax.experimental import pallas as pl
from jax.experimental.pallas import tpu as pltpu
from jax.experimental.pallas import tpu_sc as plsc

C, H, W = 9, 260, 346
NB = 8
NEV = 2_000_000
RB = NEV // NB
WH = W * H
WHC = WH * C
S = 2 * WHC
NC, NS = 2, 16
NCH = 123
GRP = 8
LROW = NS * NCH * 128
SPAD = 1_619_456
TSLICE = SPAD // NS
FL_LAST = S - (NS - 1) * TSLICE
ZCH = 4_096
NZF = TSLICE // ZCH
ZTAIL = TSLICE - NZF * ZCH
FTAIL_LAST = FL_LAST - NZF * ZCH

_BOUNDS = [np.float32(i / C) for i in range(1, C)]
NROW = NS * NCH
RBP = NROW * 128
HROW = NROW // 2

_SELT = np.zeros((640, 128), np.float32)
_SELLIN = np.zeros((640, 128), np.float32)
for _j in range(128):
    _SELT[5 * _j + 2, _j] = 1.0
    _SELLIN[5 * _j + 0, _j] = 1.0
    _SELLIN[5 * _j + 1, _j] = np.float32(W)
    _SELLIN[5 * _j + 3, _j] = np.float32(WHC)


def _dot(a, b):
    return jax.lax.dot(a, b, precision=jax.lax.Precision.HIGHEST,
                       preferred_element_type=jnp.float32)


def _tmax_body(ev_ref, selt_ref, out_ref):
    ev = ev_ref[0]
    t = _dot(ev, selt_ref[...])
    ii = (lax.broadcasted_iota(jnp.int32, (NROW, 128), 0) * 128
          + lax.broadcasted_iota(jnp.int32, (NROW, 128), 1))
    t = jnp.where(ii < RB, t, 0.0)
    out_ref[...] = jnp.full((1, 1, 128), jnp.max(t), jnp.float32)


def _idx_body(ev_ref, selt_ref, sellin_ref, tmax_ref, out_ref):
    j = pl.program_id(1)
    ev = ev_ref[0]
    t = _dot(ev, selt_ref[...])
    lin = _dot(ev, sellin_ref[...])
    tmax = jnp.max(tmax_ref[...])
    tn = t / tmax
    bin_ = jnp.zeros((HROW, 128), jnp.int32)
    for cb in _BOUNDS:
        bin_ = bin_ + (tn > cb).astype(jnp.int32)
    idx = lin.astype(jnp.int32) + WH * bin_
    ii = ((j * HROW + lax.broadcasted_iota(jnp.int32, (HROW, 128), 0)) * 128
          + lax.broadcasted_iota(jnp.int32, (HROW, 128), 1))
    idx = jnp.where(ii < RB, idx, S)
    out_ref[...] = idx[None]


def _compute_idx(evp):
    tmax = pl.pallas_call(
        _tmax_body,
        grid=(NB,),
        in_specs=[pl.BlockSpec((1, NROW, 640), lambda b: (b, 0, 0)),
                  pl.BlockSpec((640, 128), lambda b: (0, 0))],
        out_specs=pl.BlockSpec((1, 1, 128), lambda b: (b, 0, 0)),
        out_shape=jax.ShapeDtypeStruct((NB, 1, 128), jnp.float32),
    )(evp, _SELT)
    return pl.pallas_call(
        _idx_body,
        grid=(NB, 2),
        in_specs=[pl.BlockSpec((1, HROW, 640), lambda b, j: (b, j, 0)),
                  pl.BlockSpec((640, 128), lambda b, j: (0, 0)),
                  pl.BlockSpec((640, 128), lambda b, j: (0, 0)),
                  pl.BlockSpec((1, 1, 128), lambda b, j: (b, 0, 0))],
        out_specs=pl.BlockSpec((1, HROW, 128), lambda b, j: (b, j, 0)),
        out_shape=jax.ShapeDtypeStruct((NB, NROW, 128), jnp.int32),
    )(evp, _SELT, _SELLIN, tmax)


@functools.cache
def _make_sc_hist():
    mesh = plsc.VectorSubcoreMesh(
        core_axis_name="c", subcore_axis_name="s", num_cores=NC, num_subcores=NS
    )

    @functools.partial(
        pl.kernel,
        out_type=jax.ShapeDtypeStruct((NB * S,), jnp.float32),
        mesh=mesh,
        scratch_types=[
            pltpu.VMEM((NCH, 128), jnp.int32),
            pltpu.VMEM((GRP, 128), jnp.float32),
            pltpu.VMEM((ZCH,), jnp.float32),
            pltpu.VMEM((ZCH,), jnp.float32),
            pltpu.VMEM_SHARED((SPAD,), jnp.float32),
            pltpu.SemaphoreType.DMA,
        ],
    )
    def _sc_hist(idx_hbm, ones_hbm, zeros_hbm, out_hbm, idx_v, ones_v,
                 zero_v, buf_v, hist, sem):
        c = lax.axis_index("c")
        s = lax.axis_index("s")
        pltpu.sync_copy(ones_hbm, ones_v)
        pltpu.sync_copy(zeros_hbm, zero_v)
        for bl in range(NB // NC):
            b = c * (NB // NC) + bl
            zoff = s * TSLICE
            zdescs = [
                pltpu.async_copy(zero_v,
                                 hist.at[pl.ds(zoff + k * ZCH, ZCH)], sem)
                for k in range(NZF)
            ]
            zdescs.append(
                pltpu.async_copy(zero_v.at[pl.ds(0, ZTAIL)],
                                 hist.at[pl.ds(zoff + NZF * ZCH, ZTAIL)],
                                 sem))
            for d in zdescs:
                d.wait()
            plsc.subcore_barrier()
            pltpu.sync_copy(idx_hbm.at[b, s], idx_v)

            def _grp(g, carry):
                descs = [
                    pltpu.async_copy(ones_v.at[j],
                                     hist.at[idx_v.at[g * GRP + j]],
                                     sem, add=True)
                    for j in range(GRP)
                ]
                for d in descs:
                    d.wait()
                return carry

            lax.fori_loop(0, NCH // GRP, _grp, 0)
            tail = [
                pltpu.async_copy(ones_v.at[j],
                                 hist.at[idx_v.at[(NCH // GRP) * GRP + j]],
                                 sem, add=True)
                for j in range(NCH % GRP)
            ]
            for d in tail:
                d.wait()
            plsc.subcore_barrier()
            ooff = b * S + s * TSLICE

            def _flush(k, carry):
                pltpu.sync_copy(hist.at[pl.ds(zoff + k * ZCH, ZCH)], buf_v)
                pltpu.sync_copy(buf_v, out_hbm.at[pl.ds(ooff + k * ZCH, ZCH)])
                return carry

            lax.fori_loop(0, NZF, _flush, 0)

            @pl.when(s < NS - 1)
            def _():
                pltpu.sync_copy(hist.at[pl.ds(zoff + NZF * ZCH, ZTAIL)],
                                buf_v.at[pl.ds(0, ZTAIL)])
                pltpu.sync_copy(buf_v.at[pl.ds(0, ZTAIL)],
                                out_hbm.at[pl.ds(ooff + NZF * ZCH, ZTAIL)])

            @pl.when(s == NS - 1)
            def _():
                pltpu.sync_copy(hist.at[pl.ds(zoff + NZF * ZCH, FTAIL_LAST)],
                                buf_v.at[pl.ds(0, FTAIL_LAST)])
                pltpu.sync_copy(buf_v.at[pl.ds(0, FTAIL_LAST)],
                                out_hbm.at[pl.ds(ooff + NZF * ZCH, FTAIL_LAST)])

    return _sc_hist


def kernel(events):
    evp = jnp.pad(events.reshape(NB, RB * 5), ((0, 0), (0, (RBP - RB) * 5)))
    evp = evp.reshape(NB, NROW, 640)
    idxp = _compute_idx(evp)
    idx4 = idxp.reshape(NB, NS, NCH, 128)
    ones = jnp.ones((GRP, 128), jnp.float32)
    zeros = jnp.zeros((ZCH,), jnp.float32)
    vox = _make_sc_hist()(idx4, ones, zeros)
    return vox.reshape(NB, 2 * C, H, W)


# speedup vs baseline: 3.2246x; 3.2246x over previous
import functools

import jax
import jax.numpy as jnp
import numpy as np
from jax import lax
from jax.experimental import pallas as pl
from jax.experimental.pallas import tpu as pltpu
from jax.experimental.pallas import tpu_sc as plsc

C, H, W = 9, 260, 346
NB = 8
NEV = 2_000_000
RB = NEV // NB
WH = W * H
WHC = WH * C
S = 2 * WHC
NC, NS = 2, 16
NCH = 123
GRP = 8
LROW = NS * NCH * 128
SPAD = 1_619_456
TSLICE = SPAD // NS
FL_LAST = S - (NS - 1) * TSLICE
ZCH = 4_096
NZF = TSLICE // ZCH
ZTAIL = TSLICE - NZF * ZCH
FTAIL_LAST = FL_LAST - NZF * ZCH

_BOUNDS = [np.float32(i / C) for i in range(1, C)]
BL = 16_000
NBLK = NEV // BL


def _pmax_body(ev_ref, out_ref):
    g = pl.program_id(0)
    t = ev_ref[2:3, :]
    ii = g * BL + lax.broadcasted_iota(jnp.int32, (1, BL), 1)
    bi = ii // RB
    lane = lax.broadcasted_iota(jnp.int32, (1, 128), 1)
    row = jnp.zeros((1, 128), jnp.float32)
    for b in range(NB):
        mb = jnp.max(jnp.where(bi == b, t, 0.0))
        row = jnp.where(lane == b, mb, row)
    out_ref[...] = row[None]


def _idx_body(ev_ref, pmax_ref, out_ref):
    g = pl.program_id(0)
    x = ev_ref[0:1, :]
    y = ev_ref[1:2, :]
    t = ev_ref[2:3, :]
    p = ev_ref[3:4, :]
    tmax_l = jnp.max(pmax_ref[...], axis=(0, 1))
    ii = g * BL + lax.broadcasted_iota(jnp.int32, (1, BL), 1)
    bi = ii // RB
    tmax_e = jnp.zeros((1, BL), jnp.float32)
    for b in range(NB):
        tmax_e = jnp.where(bi == b, tmax_l[b], tmax_e)
    tn = t / tmax_e
    bin_ = jnp.zeros((1, BL), jnp.int32)
    for cb in _BOUNDS:
        bin_ = bin_ + (tn > cb).astype(jnp.int32)
    idx = (x.astype(jnp.int32) + W * y.astype(jnp.int32)
           + WHC * p.astype(jnp.int32) + WH * bin_)
    out_ref[...] = idx


def _compute_idx(evT):
    pmax = pl.pallas_call(
        _pmax_body,
        grid=(NBLK,),
        in_specs=[pl.BlockSpec((5, BL), lambda g: (0, g))],
        out_specs=pl.BlockSpec((1, 1, 128), lambda g: (g, 0, 0)),
        out_shape=jax.ShapeDtypeStruct((NBLK, 1, 128), jnp.float32),
    )(evT)
    return pl.pallas_call(
        _idx_body,
        grid=(NBLK,),
        in_specs=[pl.BlockSpec((5, BL), lambda g: (0, g)),
                  pl.BlockSpec((NBLK, 1, 128), lambda g: (0, 0, 0))],
        out_specs=pl.BlockSpec((1, BL), lambda g: (0, g)),
        out_shape=jax.ShapeDtypeStruct((1, NEV), jnp.int32),
    )(evT, pmax)


@functools.cache
def _make_sc_hist():
    mesh = plsc.VectorSubcoreMesh(
        core_axis_name="c", subcore_axis_name="s", num_cores=NC, num_subcores=NS
    )

    @functools.partial(
        pl.kernel,
        out_type=jax.ShapeDtypeStruct((NB * S,), jnp.float32),
        mesh=mesh,
        scratch_types=[
            pltpu.VMEM((NCH, 128), jnp.int32),
            pltpu.VMEM((GRP, 128), jnp.float32),
            pltpu.VMEM((ZCH,), jnp.float32),
            pltpu.VMEM((ZCH,), jnp.float32),
            pltpu.VMEM_SHARED((SPAD,), jnp.float32),
            pltpu.SemaphoreType.DMA,
        ],
    )
    def _sc_hist(idx_hbm, ones_hbm, zeros_hbm, out_hbm, idx_v, ones_v,
                 zero_v, buf_v, hist, sem):
        c = lax.axis_index("c")
        s = lax.axis_index("s")
        pltpu.sync_copy(ones_hbm, ones_v)
        pltpu.sync_copy(zeros_hbm, zero_v)
        for bl in range(NB // NC):
            b = c * (NB // NC) + bl
            zoff = s * TSLICE
            zdescs = [
                pltpu.async_copy(zero_v,
                                 hist.at[pl.ds(zoff + k * ZCH, ZCH)], sem)
                for k in range(NZF)
            ]
            zdescs.append(
                pltpu.async_copy(zero_v.at[pl.ds(0, ZTAIL)],
                                 hist.at[pl.ds(zoff + NZF * ZCH, ZTAIL)],
                                 sem))
            for d in zdescs:
                d.wait()
            plsc.subcore_barrier()
            pltpu.sync_copy(idx_hbm.at[b, s], idx_v)

            def _grp(g, carry):
                descs = [
                    pltpu.async_copy(ones_v.at[j],
                                     hist.at[idx_v.at[g * GRP + j]],
                                     sem, add=True)
                    for j in range(GRP)
                ]
                for d in descs:
                    d.wait()
                return carry

            lax.fori_loop(0, NCH // GRP, _grp, 0)
            tail = [
                pltpu.async_copy(ones_v.at[j],
                                 hist.at[idx_v.at[(NCH // GRP) * GRP + j]],
                                 sem, add=True)
                for j in range(NCH % GRP)
            ]
            for d in tail:
                d.wait()
            plsc.subcore_barrier()
            ooff = b * S + s * TSLICE

            def _flush(k, carry):
                pltpu.sync_copy(hist.at[pl.ds(zoff + k * ZCH, ZCH)], buf_v)
                pltpu.sync_copy(buf_v, out_hbm.at[pl.ds(ooff + k * ZCH, ZCH)])
                return carry

            lax.fori_loop(0, NZF, _flush, 0)

            @pl.when(s < NS - 1)
            def _():
                pltpu.sync_copy(hist.at[pl.ds(zoff + NZF * ZCH, ZTAIL)],
                                buf_v.at[pl.ds(0, ZTAIL)])
                pltpu.sync_copy(buf_v.at[pl.ds(0, ZTAIL)],
                                out_hbm.at[pl.ds(ooff + NZF * ZCH, ZTAIL)])

            @pl.when(s == NS - 1)
            def _():
                pltpu.sync_copy(hist.at[pl.ds(zoff + NZF * ZCH, FTAIL_LAST)],
                                buf_v.at[pl.ds(0, FTAIL_LAST)])
                pltpu.sync_copy(buf_v.at[pl.ds(0, FTAIL_LAST)],
                                out_hbm.at[pl.ds(ooff + NZF * ZCH, FTAIL_LAST)])

    return _sc_hist


def kernel(events):
    evT = events.T
    idx = _compute_idx(evT).reshape(NB, RB)
    idxp = jnp.pad(idx, ((0, 0), (0, NS * NCH * 128 - RB)), constant_values=S)
    idx4 = idxp.reshape(NB, NS, NCH, 128)
    ones = jnp.ones((GRP, 128), jnp.float32)
    zeros = jnp.zeros((ZCH,), jnp.float32)
    vox = _make_sc_hist()(idx4, ones, zeros)
    return vox.reshape(NB, 2 * C, H, W)


# speedup vs baseline: 3.9759x; 1.2330x over previous
import functools

import jax
import jax.numpy as jnp
import numpy as np
from jax import lax
from jax.experimental import pallas as pl
from jax.experimental.pallas import tpu as pltpu
from jax.experimental.pallas import tpu_sc as plsc

C, H, W = 9, 260, 346
NB = 8
NEV = 2_000_000
RB = NEV // NB
WH = W * H
WHC = WH * C
S = 2 * WHC
NC, NS = 2, 16
NCH = 123
GRP = 8
LROW = NS * NCH * 128
SPAD = 1_619_456
TSLICE = SPAD // NS
FL_LAST = S - (NS - 1) * TSLICE
ZCH = 4_096
NZF = TSLICE // ZCH
ZTAIL = TSLICE - NZF * ZCH
FTAIL_LAST = FL_LAST - NZF * ZCH

_BOUNDS = [np.float32(i / C) for i in range(1, C)]
BL = 16_000
NBLK = NEV // BL


def _pmax_body(ev_ref, out_ref):
    g = pl.program_id(0)
    t = ev_ref[2:3, :]
    b0 = (g * BL) // RB
    thresh = (b0 + 1) * RB
    ii = g * BL + lax.broadcasted_iota(jnp.int32, (1, BL), 1)
    m0 = ii < thresh
    mx0 = jnp.max(jnp.where(m0, t, 0.0))
    mx1 = jnp.max(jnp.where(m0, 0.0, t))
    lane = lax.broadcasted_iota(jnp.int32, (1, 128), 1)
    row = (jnp.where(lane == b0, mx0, 0.0)
           + jnp.where(lane == b0 + 1, mx1, 0.0))
    out_ref[...] = row[None]


def _idx_body(ev_ref, pmax_ref, out_ref):
    g = pl.program_id(0)
    x = ev_ref[0:1, :]
    y = ev_ref[1:2, :]
    t = ev_ref[2:3, :]
    p = ev_ref[3:4, :]
    tmax_row = jnp.max(pmax_ref[...], axis=0)
    b0 = (g * BL) // RB
    thresh = (b0 + 1) * RB
    lane = lax.broadcasted_iota(jnp.int32, (1, 128), 1)
    tl0 = jnp.max(jnp.where(lane == b0, tmax_row, -1.0))
    tl1 = jnp.max(jnp.where(lane == b0 + 1, tmax_row, -1.0))
    ii = g * BL + lax.broadcasted_iota(jnp.int32, (1, BL), 1)
    tmax_e = jnp.where(ii < thresh, tl0, tl1)
    tn = t / tmax_e
    bin_ = jnp.zeros((1, BL), jnp.int32)
    for cb in _BOUNDS:
        bin_ = bin_ + (tn > cb).astype(jnp.int32)
    idx = (x.astype(jnp.int32) + W * y.astype(jnp.int32)
           + WHC * p.astype(jnp.int32) + WH * bin_)
    out_ref[...] = idx[None]


def _compute_idx(evT):
    pmax = pl.pallas_call(
        _pmax_body,
        grid=(NBLK,),
        in_specs=[pl.BlockSpec((5, BL), lambda g: (0, g))],
        out_specs=pl.BlockSpec((1, 1, 128), lambda g: (g, 0, 0)),
        out_shape=jax.ShapeDtypeStruct((NBLK, 1, 128), jnp.float32),
    )(evT)
    return pl.pallas_call(
        _idx_body,
        grid=(NBLK,),
        in_specs=[pl.BlockSpec((5, BL), lambda g: (0, g)),
                  pl.BlockSpec((NBLK, 1, 128), lambda g: (0, 0, 0))],
        out_specs=pl.BlockSpec((1, 1, BL), lambda g: (g, 0, 0)),
        out_shape=jax.ShapeDtypeStruct((NBLK, 1, BL), jnp.int32),
    )(evT, pmax)


@functools.cache
def _make_sc_hist():
    mesh = plsc.VectorSubcoreMesh(
        core_axis_name="c", subcore_axis_name="s", num_cores=NC, num_subcores=NS
    )

    @functools.partial(
        pl.kernel,
        out_type=jax.ShapeDtypeStruct((NB * S,), jnp.float32),
        mesh=mesh,
        scratch_types=[
            pltpu.VMEM((NCH, 128), jnp.int32),
            pltpu.VMEM((GRP, 128), jnp.float32),
            pltpu.VMEM((ZCH,), jnp.float32),
            pltpu.VMEM((ZCH,), jnp.float32),
            pltpu.VMEM_SHARED((SPAD,), jnp.float32),
            pltpu.SemaphoreType.DMA,
        ],
    )
    def _sc_hist(idx_hbm, ones_hbm, zeros_hbm, out_hbm, idx_v, ones_v,
                 zero_v, buf_v, hist, sem):
        c = lax.axis_index("c")
        s = lax.axis_index("s")
        pltpu.sync_copy(ones_hbm, ones_v)
        pltpu.sync_copy(zeros_hbm, zero_v)
        for bl in range(NB // NC):
            b = c * (NB // NC) + bl
            zoff = s * TSLICE
            zdescs = [
                pltpu.async_copy(zero_v,
                                 hist.at[pl.ds(zoff + k * ZCH, ZCH)], sem)
                for k in range(NZF)
            ]
            zdescs.append(
                pltpu.async_copy(zero_v.at[pl.ds(0, ZTAIL)],
                                 hist.at[pl.ds(zoff + NZF * ZCH, ZTAIL)],
                                 sem))
            for d in zdescs:
                d.wait()
            plsc.subcore_barrier()
            pltpu.sync_copy(idx_hbm.at[b, s], idx_v)

            def _grp(g, carry):
                descs = [
                    pltpu.async_copy(ones_v.at[j],
                                     hist.at[idx_v.at[g * GRP + j]],
                                     sem, add=True)
                    for j in range(GRP)
                ]
                for d in descs:
                    d.wait()
                return carry

            lax.fori_loop(0, NCH // GRP, _grp, 0)
            tail = [
                pltpu.async_copy(ones_v.at[j],
                                 hist.at[idx_v.at[(NCH // GRP) * GRP + j]],
                                 sem, add=True)
                for j in range(NCH % GRP)
            ]
            for d in tail:
                d.wait()
            plsc.subcore_barrier()
            ooff = b * S + s * TSLICE

            def _flush(k, carry):
                pltpu.sync_copy(hist.at[pl.ds(zoff + k * ZCH, ZCH)], buf_v)
                pltpu.sync_copy(buf_v, out_hbm.at[pl.ds(ooff + k * ZCH, ZCH)])
                return carry

            lax.fori_loop(0, NZF, _flush, 0)

            @pl.when(s < NS - 1)
            def _():
                pltpu.sync_copy(hist.at[pl.ds(zoff + NZF * ZCH, ZTAIL)],
                                buf_v.at[pl.ds(0, ZTAIL)])
                pltpu.sync_copy(buf_v.at[pl.ds(0, ZTAIL)],
                                out_hbm.at[pl.ds(ooff + NZF * ZCH, ZTAIL)])

            @pl.when(s == NS - 1)
            def _():
                pltpu.sync_copy(hist.at[pl.ds(zoff + NZF * ZCH, FTAIL_LAST)],
                                buf_v.at[pl.ds(0, FTAIL_LAST)])
                pltpu.sync_copy(buf_v.at[pl.ds(0, FTAIL_LAST)],
                                out_hbm.at[pl.ds(ooff + NZF * ZCH, FTAIL_LAST)])

    return _sc_hist


def kernel(events):
    evT = events.T
    idx = _compute_idx(evT).reshape(NB, RB)
    idxp = jnp.pad(idx, ((0, 0), (0, NS * NCH * 128 - RB)), constant_values=S)
    idx4 = idxp.reshape(NB, NS, NCH, 128)
    ones = jnp.ones((GRP, 128), jnp.float32)
    zeros = jnp.zeros((ZCH,), jnp.float32)
    vox = _make_sc_hist()(idx4, ones, zeros)
    return vox.reshape(NB, 2 * C, H, W)


# speedup vs baseline: 4.5121x; 1.1348x over previous
import functools

import jax
import jax.numpy as jnp
import numpy as np
from jax import lax
from jax.experimental import pallas as pl
from jax.experimental.pallas import tpu as pltpu
from jax.experimental.pallas import tpu_sc as plsc

C, H, W = 9, 260, 346
NB = 8
NEV = 2_000_000
RB = NEV // NB
WH = W * H
WHC = WH * C
S = 2 * WHC
NC, NS = 2, 16
NCH = 123
GRP = 8
LROW = NS * NCH * 128
SPAD = 1_619_456
TSLICE = SPAD // NS
FL_LAST = S - (NS - 1) * TSLICE
ZCH = 4_096
NZF = TSLICE // ZCH
ZTAIL = TSLICE - NZF * ZCH
FTAIL_LAST = FL_LAST - NZF * ZCH

_BOUNDS = [np.float32(i / C) for i in range(1, C)]
BL = 16_000
NBLK = NEV // BL


def _pmax_body(ev_ref, out_ref):
    g = pl.program_id(0)
    t = ev_ref[2:3, :]
    b0 = (g * BL) // RB
    thresh = (b0 + 1) * RB
    ii = g * BL + lax.broadcasted_iota(jnp.int32, (1, BL), 1)
    m0 = ii < thresh
    mx0 = jnp.max(jnp.where(m0, t, 0.0))
    mx1 = jnp.max(jnp.where(m0, 0.0, t))
    lane = lax.broadcasted_iota(jnp.int32, (1, 128), 1)
    row = (jnp.where(lane == b0, mx0, 0.0)
           + jnp.where(lane == b0 + 1, mx1, 0.0))
    out_ref[...] = row[None]


def _idx_body(ev_ref, pmax_ref, out_ref):
    g = pl.program_id(0)
    x = ev_ref[0:1, :]
    y = ev_ref[1:2, :]
    t = ev_ref[2:3, :]
    p = ev_ref[3:4, :]
    tmax_row = jnp.max(pmax_ref[...], axis=0)
    b0 = (g * BL) // RB
    thresh = (b0 + 1) * RB
    lane = lax.broadcasted_iota(jnp.int32, (1, 128), 1)
    tl0 = jnp.max(jnp.where(lane == b0, tmax_row, -1.0))
    tl1 = jnp.max(jnp.where(lane == b0 + 1, tmax_row, -1.0))
    ii = g * BL + lax.broadcasted_iota(jnp.int32, (1, BL), 1)
    tmax_e = jnp.where(ii < thresh, tl0, tl1)
    tn = t / tmax_e
    bin_ = jnp.zeros((1, BL), jnp.int32)
    for cb in _BOUNDS:
        bin_ = bin_ + (tn > cb).astype(jnp.int32)
    idx = (x.astype(jnp.int32) + W * y.astype(jnp.int32)
           + WHC * p.astype(jnp.int32) + WH * bin_)
    out_ref[...] = idx.reshape(1, 1, BL // 128, 128)


def _compute_idx(evT):
    pmax = pl.pallas_call(
        _pmax_body,
        grid=(NBLK,),
        in_specs=[pl.BlockSpec((5, BL), lambda g: (0, g))],
        out_specs=pl.BlockSpec((1, 1, 128), lambda g: (g, 0, 0)),
        out_shape=jax.ShapeDtypeStruct((NBLK, 1, 128), jnp.float32),
    )(evT)
    return pl.pallas_call(
        _idx_body,
        grid=(NBLK,),
        in_specs=[pl.BlockSpec((5, BL), lambda g: (0, g)),
                  pl.BlockSpec((NBLK, 1, 128), lambda g: (0, 0, 0))],
        out_specs=pl.BlockSpec((1, 1, BL // 128, 128), lambda g: (g, 0, 0, 0)),
        out_shape=jax.ShapeDtypeStruct((NBLK, 1, BL // 128, 128), jnp.int32),
    )(evT, pmax)


@functools.cache
def _make_sc_hist():
    mesh = plsc.VectorSubcoreMesh(
        core_axis_name="c", subcore_axis_name="s", num_cores=NC, num_subcores=NS
    )

    @functools.partial(
        pl.kernel,
        out_type=jax.ShapeDtypeStruct((NB * S,), jnp.float32),
        mesh=mesh,
        scratch_types=[
            pltpu.VMEM((NCH, 128), jnp.int32),
            pltpu.VMEM((GRP, 128), jnp.float32),
            pltpu.VMEM((ZCH,), jnp.float32),
            pltpu.VMEM((ZCH,), jnp.float32),
            pltpu.VMEM_SHARED((SPAD,), jnp.float32),
            pltpu.SemaphoreType.DMA,
        ],
    )
    def _sc_hist(idx_hbm, ones_hbm, zeros_hbm, out_hbm, idx_v, ones_v,
                 zero_v, buf_v, hist, sem):
        c = lax.axis_index("c")
        s = lax.axis_index("s")
        pltpu.sync_copy(ones_hbm, ones_v)
        pltpu.sync_copy(zeros_hbm, zero_v)
        for bl in range(NB // NC):
            b = c * (NB // NC) + bl
            zoff = s * TSLICE
            zdescs = [
                pltpu.async_copy(zero_v,
                                 hist.at[pl.ds(zoff + k * ZCH, ZCH)], sem)
                for k in range(NZF)
            ]
            zdescs.append(
                pltpu.async_copy(zero_v.at[pl.ds(0, ZTAIL)],
                                 hist.at[pl.ds(zoff + NZF * ZCH, ZTAIL)],
                                 sem))
            for d in zdescs:
                d.wait()
            plsc.subcore_barrier()
            pltpu.sync_copy(idx_hbm.at[b, s], idx_v)

            def _grp(g, carry):
                descs = [
                    pltpu.async_copy(ones_v.at[j],
                                     hist.at[idx_v.at[g * GRP + j]],
                                     sem, add=True)
                    for j in range(GRP)
                ]
                for d in descs:
                    d.wait()
                return carry

            lax.fori_loop(0, NCH // GRP, _grp, 0)
            tail = [
                pltpu.async_copy(ones_v.at[j],
                                 hist.at[idx_v.at[(NCH // GRP) * GRP + j]],
                                 sem, add=True)
                for j in range(NCH % GRP)
            ]
            for d in tail:
                d.wait()
            plsc.subcore_barrier()
            ooff = b * S + s * TSLICE

            def _flush(k, carry):
                pltpu.sync_copy(hist.at[pl.ds(zoff + k * ZCH, ZCH)], buf_v)
                pltpu.sync_copy(buf_v, out_hbm.at[pl.ds(ooff + k * ZCH, ZCH)])
                return carry

            lax.fori_loop(0, NZF, _flush, 0)

            @pl.when(s < NS - 1)
            def _():
                pltpu.sync_copy(hist.at[pl.ds(zoff + NZF * ZCH, ZTAIL)],
                                buf_v.at[pl.ds(0, ZTAIL)])
                pltpu.sync_copy(buf_v.at[pl.ds(0, ZTAIL)],
                                out_hbm.at[pl.ds(ooff + NZF * ZCH, ZTAIL)])

            @pl.when(s == NS - 1)
            def _():
                pltpu.sync_copy(hist.at[pl.ds(zoff + NZF * ZCH, FTAIL_LAST)],
                                buf_v.at[pl.ds(0, FTAIL_LAST)])
                pltpu.sync_copy(buf_v.at[pl.ds(0, FTAIL_LAST)],
                                out_hbm.at[pl.ds(ooff + NZF * ZCH, FTAIL_LAST)])

    return _sc_hist


def kernel(events):
    evT = events.T
    idx = _compute_idx(evT).reshape(NB, RB)
    idxp = jnp.pad(idx, ((0, 0), (0, NS * NCH * 128 - RB)), constant_values=S)
    idx4 = idxp.reshape(NB, NS, NCH, 128)
    ones = jnp.ones((GRP, 128), jnp.float32)
    zeros = jnp.zeros((ZCH,), jnp.float32)
    vox = _make_sc_hist()(idx4, ones, zeros)
    return vox.reshape(NB, 2 * C, H, W)


# speedup vs baseline: 4.6178x; 1.0234x over previous
import functools

import jax
import jax.numpy as jnp
import numpy as np
from jax import lax
from jax.experimental import pallas as pl
from jax.experimental.pallas import tpu as pltpu
from jax.experimental.pallas import tpu_sc as plsc

C, H, W = 9, 260, 346
NB = 8
NEV = 2_000_000
RB = NEV // NB
WH = W * H
WHC = WH * C
S = 2 * WHC
NC, NS = 2, 16
NCH = 123
GRP = 16
LROW = NS * NCH * 128
SPAD = 1_619_456
TSLICE = SPAD // NS
FL_LAST = S - (NS - 1) * TSLICE
ZCH = 4_096
NZF = TSLICE // ZCH
ZTAIL = TSLICE - NZF * ZCH
FTAIL_LAST = FL_LAST - NZF * ZCH

_BOUNDS = [np.float32(i / C) for i in range(1, C)]
BL = 16_000
NBLK = NEV // BL


def _pmax_body(ev_ref, out_ref):
    g = pl.program_id(0)
    t = ev_ref[2:3, :]
    b0 = (g * BL) // RB
    thresh = (b0 + 1) * RB
    ii = g * BL + lax.broadcasted_iota(jnp.int32, (1, BL), 1)
    m0 = ii < thresh
    mx0 = jnp.max(jnp.where(m0, t, 0.0))
    mx1 = jnp.max(jnp.where(m0, 0.0, t))
    lane = lax.broadcasted_iota(jnp.int32, (1, 128), 1)
    row = (jnp.where(lane == b0, mx0, 0.0)
           + jnp.where(lane == b0 + 1, mx1, 0.0))
    out_ref[...] = row[None]


def _idx_body(ev_ref, pmax_ref, out_ref):
    g = pl.program_id(0)
    x = ev_ref[0:1, :]
    y = ev_ref[1:2, :]
    t = ev_ref[2:3, :]
    p = ev_ref[3:4, :]
    tmax_row = jnp.max(pmax_ref[...], axis=0)
    b0 = (g * BL) // RB
    thresh = (b0 + 1) * RB
    lane = lax.broadcasted_iota(jnp.int32, (1, 128), 1)
    tl0 = jnp.max(jnp.where(lane == b0, tmax_row, -1.0))
    tl1 = jnp.max(jnp.where(lane == b0 + 1, tmax_row, -1.0))
    ii = g * BL + lax.broadcasted_iota(jnp.int32, (1, BL), 1)
    tmax_e = jnp.where(ii < thresh, tl0, tl1)
    tn = t / tmax_e
    bin_ = jnp.zeros((1, BL), jnp.int32)
    for cb in _BOUNDS:
        bin_ = bin_ + (tn > cb).astype(jnp.int32)
    idx = (x.astype(jnp.int32) + W * y.astype(jnp.int32)
           + WHC * p.astype(jnp.int32) + WH * bin_)
    out_ref[...] = idx.reshape(1, 1, BL // 128, 128)


def _compute_idx(evT):
    pmax = pl.pallas_call(
        _pmax_body,
        grid=(NBLK,),
        in_specs=[pl.BlockSpec((5, BL), lambda g: (0, g))],
        out_specs=pl.BlockSpec((1, 1, 128), lambda g: (g, 0, 0)),
        out_shape=jax.ShapeDtypeStruct((NBLK, 1, 128), jnp.float32),
    )(evT)
    return pl.pallas_call(
        _idx_body,
        grid=(NBLK,),
        in_specs=[pl.BlockSpec((5, BL), lambda g: (0, g)),
                  pl.BlockSpec((NBLK, 1, 128), lambda g: (0, 0, 0))],
        out_specs=pl.BlockSpec((1, 1, BL // 128, 128), lambda g: (g, 0, 0, 0)),
        out_shape=jax.ShapeDtypeStruct((NBLK, 1, BL // 128, 128), jnp.int32),
    )(evT, pmax)


@functools.cache
def _make_sc_hist():
    mesh = plsc.VectorSubcoreMesh(
        core_axis_name="c", subcore_axis_name="s", num_cores=NC, num_subcores=NS
    )

    @functools.partial(
        pl.kernel,
        out_type=jax.ShapeDtypeStruct((NB * S,), jnp.float32),
        mesh=mesh,
        scratch_types=[
            pltpu.VMEM((NCH, 128), jnp.int32),
            pltpu.VMEM((1, 128), jnp.float32),
            pltpu.VMEM((ZCH,), jnp.float32),
            pltpu.VMEM((ZCH,), jnp.float32),
            pltpu.VMEM((ZCH,), jnp.float32),
            pltpu.VMEM_SHARED((SPAD,), jnp.float32),
            pltpu.SemaphoreType.DMA,
            pltpu.SemaphoreType.DMA,
            pltpu.SemaphoreType.DMA,
            pltpu.SemaphoreType.DMA,
        ],
    )
    def _sc_hist(idx_hbm, ones_hbm, zeros_hbm, out_hbm, idx_v, ones_v,
                 zero_v, buf0_v, buf1_v, hist, sem, sema, semz, semb):
        c = lax.axis_index("c")
        s = lax.axis_index("s")
        bufs = (buf0_v, buf1_v)
        pltpu.sync_copy(ones_hbm, ones_v)
        pltpu.sync_copy(zeros_hbm, zero_v)
        zoff = s * TSLICE
        for bl in range(NB // NC):
            b = c * (NB // NC) + bl
            if bl == 0:
                zdescs = [
                    pltpu.async_copy(zero_v,
                                     hist.at[pl.ds(zoff + k * ZCH, ZCH)],
                                     semz)
                    for k in range(NZF)
                ]
                zdescs.append(
                    pltpu.async_copy(zero_v.at[pl.ds(0, ZTAIL)],
                                     hist.at[pl.ds(zoff + NZF * ZCH, ZTAIL)],
                                     semz))
                for d in zdescs:
                    d.wait()
                plsc.subcore_barrier()
            pltpu.sync_copy(idx_hbm.at[b, s], idx_v)

            def _grp(g, carry):
                descs = [
                    pltpu.async_copy(ones_v.at[0],
                                     hist.at[idx_v.at[g * GRP + j]],
                                     sem, add=True)
                    for j in range(GRP)
                ]
                for d in descs:
                    d.wait()
                return carry

            lax.fori_loop(0, NCH // GRP, _grp, 0)
            tail = [
                pltpu.async_copy(ones_v.at[0],
                                 hist.at[idx_v.at[(NCH // GRP) * GRP + j]],
                                 sem, add=True)
                for j in range(NCH % GRP)
            ]
            for d in tail:
                d.wait()
            plsc.subcore_barrier()
            ooff = b * S + s * TSLICE
            do_zero = bl != NB // NC - 1
            zdescs = []
            for kk in range(NZF // 2):
                hd = []
                for half in range(2):
                    k = 2 * kk + half
                    d = pltpu.async_copy(hist.at[pl.ds(zoff + k * ZCH, ZCH)],
                                         bufs[half], sema)
                    d.wait()
                    if do_zero:
                        zdescs.append(
                            pltpu.async_copy(zero_v,
                                             hist.at[pl.ds(zoff + k * ZCH,
                                                           ZCH)], semz))
                    hd.append(
                        pltpu.async_copy(bufs[half],
                                         out_hbm.at[pl.ds(ooff + k * ZCH,
                                                          ZCH)], semb))
                for d in hd:
                    d.wait()

            @pl.when(s < NS - 1)
            def _():
                pltpu.sync_copy(hist.at[pl.ds(zoff + NZF * ZCH, ZTAIL)],
                                buf0_v.at[pl.ds(0, ZTAIL)])
                pltpu.sync_copy(buf0_v.at[pl.ds(0, ZTAIL)],
                                out_hbm.at[pl.ds(ooff + NZF * ZCH, ZTAIL)])

            @pl.when(s == NS - 1)
            def _():
                pltpu.sync_copy(hist.at[pl.ds(zoff + NZF * ZCH, FTAIL_LAST)],
                                buf0_v.at[pl.ds(0, FTAIL_LAST)])
                pltpu.sync_copy(buf0_v.at[pl.ds(0, FTAIL_LAST)],
                                out_hbm.at[pl.ds(ooff + NZF * ZCH, FTAIL_LAST)])

            if do_zero:
                zdescs.append(
                    pltpu.async_copy(zero_v.at[pl.ds(0, ZTAIL)],
                                     hist.at[pl.ds(zoff + NZF * ZCH, ZTAIL)],
                                     semz))
                for d in zdescs:
                    d.wait()
                plsc.subcore_barrier()

    return _sc_hist


def kernel(events):
    evT = events.T
    idx = _compute_idx(evT).reshape(NB, RB)
    idxp = jnp.pad(idx, ((0, 0), (0, NS * NCH * 128 - RB)), constant_values=S)
    idx4 = idxp.reshape(NB, NS, NCH, 128)
    ones = jnp.ones((1, 128), jnp.float32)
    zeros = jnp.zeros((ZCH,), jnp.float32)
    vox = _make_sc_hist()(idx4, ones, zeros)
    return vox.reshape(NB, 2 * C, H, W)


# speedup vs baseline: 5.5487x; 1.2016x over previous
import functools

import jax
import jax.numpy as jnp
import numpy as np
from jax import lax
from jax.experimental import pallas as pl
from jax.experimental.pallas import tpu as pltpu
from jax.experimental.pallas import tpu_sc as plsc

C, H, W = 9, 260, 346
NB = 8
NEV = 2_000_000
RB = NEV // NB
WH = W * H
WHC = WH * C
S = 2 * WHC
NC, NS = 2, 16
NCH = 123
GRP = 16
LROW = NS * NCH * 128
SPAD = 1_619_456
TSLICE = SPAD // NS
FL_LAST = S - (NS - 1) * TSLICE
ZCH = 4_096
NZF = TSLICE // ZCH
ZTAIL = TSLICE - NZF * ZCH
FTAIL_LAST = FL_LAST - NZF * ZCH

_BOUNDS = [np.float32(i / C) for i in range(1, C)]
BL = 80_000
NBLK = NEV // BL


def _pmax_body(ev_ref, out_ref):
    g = pl.program_id(0)
    t = ev_ref[2:3, :]
    b0 = (g * BL) // RB
    thresh = (b0 + 1) * RB
    ii = g * BL + lax.broadcasted_iota(jnp.int32, (1, BL), 1)
    m0 = ii < thresh
    mx0 = jnp.max(jnp.where(m0, t, 0.0))
    mx1 = jnp.max(jnp.where(m0, 0.0, t))
    lane = lax.broadcasted_iota(jnp.int32, (1, 128), 1)
    row = (jnp.where(lane == b0, mx0, 0.0)
           + jnp.where(lane == b0 + 1, mx1, 0.0))
    out_ref[...] = row[None]


def _idx_body(ev_ref, pmax_ref, out_ref):
    g = pl.program_id(0)
    x = ev_ref[0:1, :]
    y = ev_ref[1:2, :]
    t = ev_ref[2:3, :]
    p = ev_ref[3:4, :]
    tmax_row = jnp.max(pmax_ref[...], axis=0)
    b0 = (g * BL) // RB
    thresh = (b0 + 1) * RB
    lane = lax.broadcasted_iota(jnp.int32, (1, 128), 1)
    tl0 = jnp.max(jnp.where(lane == b0, tmax_row, -1.0))
    tl1 = jnp.max(jnp.where(lane == b0 + 1, tmax_row, -1.0))
    ii = g * BL + lax.broadcasted_iota(jnp.int32, (1, BL), 1)
    tmax_e = jnp.where(ii < thresh, tl0, tl1)
    tn = t / tmax_e
    bin_ = jnp.zeros((1, BL), jnp.int32)
    for cb in _BOUNDS:
        bin_ = bin_ + (tn > cb).astype(jnp.int32)
    idx = (x.astype(jnp.int32) + W * y.astype(jnp.int32)
           + WHC * p.astype(jnp.int32) + WH * bin_)
    out_ref[...] = idx.reshape(1, 1, BL // 128, 128)


def _compute_idx(evT):
    pmax = pl.pallas_call(
        _pmax_body,
        grid=(NBLK,),
        in_specs=[pl.BlockSpec((5, BL), lambda g: (0, g))],
        out_specs=pl.BlockSpec((1, 1, 128), lambda g: (g, 0, 0)),
        out_shape=jax.ShapeDtypeStruct((NBLK, 1, 128), jnp.float32),
    )(evT)
    return pl.pallas_call(
        _idx_body,
        grid=(NBLK,),
        in_specs=[pl.BlockSpec((5, BL), lambda g: (0, g)),
                  pl.BlockSpec((NBLK, 1, 128), lambda g: (0, 0, 0))],
        out_specs=pl.BlockSpec((1, 1, BL // 128, 128), lambda g: (g, 0, 0, 0)),
        out_shape=jax.ShapeDtypeStruct((NBLK, 1, BL // 128, 128), jnp.int32),
    )(evT, pmax)


@functools.cache
def _make_sc_hist():
    mesh = plsc.VectorSubcoreMesh(
        core_axis_name="c", subcore_axis_name="s", num_cores=NC, num_subcores=NS
    )

    @functools.partial(
        pl.kernel,
        out_type=jax.ShapeDtypeStruct((NB * S,), jnp.float32),
        mesh=mesh,
        scratch_types=[
            pltpu.VMEM((NCH, 128), jnp.int32),
            pltpu.VMEM((1, 128), jnp.float32),
            pltpu.VMEM((ZCH,), jnp.float32),
            pltpu.VMEM((ZCH,), jnp.float32),
            pltpu.VMEM((ZCH,), jnp.float32),
            pltpu.VMEM_SHARED((SPAD,), jnp.float32),
            pltpu.SemaphoreType.DMA,
            pltpu.SemaphoreType.DMA,
            pltpu.SemaphoreType.DMA,
            pltpu.SemaphoreType.DMA,
        ],
    )
    def _sc_hist(idx_hbm, ones_hbm, zeros_hbm, out_hbm, idx_v, ones_v,
                 zero_v, buf0_v, buf1_v, hist, sem, sema, semz, semb):
        c = lax.axis_index("c")
        s = lax.axis_index("s")
        bufs = (buf0_v, buf1_v)
        pltpu.sync_copy(ones_hbm, ones_v)
        pltpu.sync_copy(zeros_hbm, zero_v)
        zoff = s * TSLICE
        for bl in range(NB // NC):
            b = c * (NB // NC) + bl
            if bl == 0:
                zdescs = [
                    pltpu.async_copy(zero_v,
                                     hist.at[pl.ds(zoff + k * ZCH, ZCH)],
                                     semz)
                    for k in range(NZF)
                ]
                zdescs.append(
                    pltpu.async_copy(zero_v.at[pl.ds(0, ZTAIL)],
                                     hist.at[pl.ds(zoff + NZF * ZCH, ZTAIL)],
                                     semz))
                for d in zdescs:
                    d.wait()
                plsc.subcore_barrier()
            pltpu.sync_copy(idx_hbm.at[b, s], idx_v)

            def _grp(g, carry):
                descs = [
                    pltpu.async_copy(ones_v.at[0],
                                     hist.at[idx_v.at[g * GRP + j]],
                                     sem, add=True)
                    for j in range(GRP)
                ]
                for d in descs:
                    d.wait()
                return carry

            lax.fori_loop(0, NCH // GRP, _grp, 0)
            tail = [
                pltpu.async_copy(ones_v.at[0],
                                 hist.at[idx_v.at[(NCH // GRP) * GRP + j]],
                                 sem, add=True)
                for j in range(NCH % GRP)
            ]
            for d in tail:
                d.wait()
            plsc.subcore_barrier()
            ooff = b * S + s * TSLICE
            do_zero = bl != NB // NC - 1
            zdescs = []
            for kk in range(NZF // 2):
                hd = []
                for half in range(2):
                    k = 2 * kk + half
                    d = pltpu.async_copy(hist.at[pl.ds(zoff + k * ZCH, ZCH)],
                                         bufs[half], sema)
                    d.wait()
                    if do_zero:
                        zdescs.append(
                            pltpu.async_copy(zero_v,
                                             hist.at[pl.ds(zoff + k * ZCH,
                                                           ZCH)], semz))
                    hd.append(
                        pltpu.async_copy(bufs[half],
                                         out_hbm.at[pl.ds(ooff + k * ZCH,
                                                          ZCH)], semb))
                for d in hd:
                    d.wait()

            @pl.when(s < NS - 1)
            def _():
                pltpu.sync_copy(hist.at[pl.ds(zoff + NZF * ZCH, ZTAIL)],
                                buf0_v.at[pl.ds(0, ZTAIL)])
                pltpu.sync_copy(buf0_v.at[pl.ds(0, ZTAIL)],
                                out_hbm.at[pl.ds(ooff + NZF * ZCH, ZTAIL)])

            @pl.when(s == NS - 1)
            def _():
                pltpu.sync_copy(hist.at[pl.ds(zoff + NZF * ZCH, FTAIL_LAST)],
                                buf0_v.at[pl.ds(0, FTAIL_LAST)])
                pltpu.sync_copy(buf0_v.at[pl.ds(0, FTAIL_LAST)],
                                out_hbm.at[pl.ds(ooff + NZF * ZCH, FTAIL_LAST)])

            if do_zero:
                zdescs.append(
                    pltpu.async_copy(zero_v.at[pl.ds(0, ZTAIL)],
                                     hist.at[pl.ds(zoff + NZF * ZCH, ZTAIL)],
                                     semz))
                for d in zdescs:
                    d.wait()
                plsc.subcore_barrier()

    return _sc_hist


def kernel(events):
    evT = events.T
    idx = _compute_idx(evT).reshape(NB, RB)
    idxp = jnp.pad(idx, ((0, 0), (0, NS * NCH * 128 - RB)), constant_values=S)
    idx4 = idxp.reshape(NB, NS, NCH, 128)
    ones = jnp.ones((1, 128), jnp.float32)
    zeros = jnp.zeros((ZCH,), jnp.float32)
    vox = _make_sc_hist()(idx4, ones, zeros)
    return vox.reshape(NB, 2 * C, H, W)
